# Initial kernel scaffold; baseline (speedup 1.0000x reference)
import jax
import jax.numpy as jnp
from jax.experimental import pallas as pl


def kernel(x, W1, gamma1, beta1, W2, gamma2, beta2, idx):
    raise NotImplementedError("write your pallas kernel here")


# speedup vs baseline: 4.6244x; 4.6244x over previous
import functools

import jax
import jax.numpy as jnp
from jax import lax
from jax.experimental import pallas as pl
from jax.experimental.pallas import tpu as pltpu
from jax.experimental.pallas import tpu_sc as plsc

_N = 10000
_K = 16
_C = 128
_H = 128
_O = 128
_EPS = 1e-5

_TILE = 200
_NT = _N // _TILE

_NW = 32
_PER_W = _N * _K // _NW
_CH = 200
_NCH = _PER_W // _CH


def _prep_body(x_ref, w1_ref, a_ref, p_ref):
    xb = x_ref[...]
    wa = w1_ref[:_C, :]
    wb = w1_ref[_C:, :]
    a_ref[...] = jnp.dot(xb, wa, preferred_element_type=jnp.float32)
    p_ref[...] = jnp.dot(xb, wb - wa, preferred_element_type=jnp.float32)


@functools.cache
def _sc_gather_fn():
    mesh = plsc.VectorSubcoreMesh(core_axis_name="c", subcore_axis_name="s")

    @functools.partial(
        pl.kernel,
        mesh=mesh,
        out_type=jax.ShapeDtypeStruct((_N * _K, _H), jnp.float32),
        scratch_types=[
            pltpu.VMEM((_PER_W,), jnp.int32),
            pltpu.VMEM((_CH, _H), jnp.float32),
            pltpu.SemaphoreType.DMA,
        ],
    )
    def _sc_gather(table_hbm, idx_hbm, out_hbm, idx_v, buf, sem):
        wid = lax.axis_index("s") * 2 + lax.axis_index("c")
        base = wid * _PER_W
        pltpu.sync_copy(idx_hbm.at[pl.ds(base, _PER_W)], idx_v)
        for i in range(_NCH):
            pltpu.async_copy(
                table_hbm.at[idx_v.at[pl.ds(i * _CH, _CH)]], buf, sem).wait()
            pltpu.sync_copy(buf, out_hbm.at[pl.ds(base + i * _CH, _CH)])

    return _sc_gather


def _stats_body(g_ref, p_ref, out_ref):
    i = pl.program_id(0)
    g = g_ref[...]
    p = p_ref[...]
    t = jnp.sum(g, axis=1)
    rows = jnp.stack([
        jnp.sum(t, axis=0),
        jnp.sum(g * g, axis=(0, 1)),
        jnp.sum(p * t, axis=0),
        jnp.sum(p, axis=0),
        jnp.sum(p * p, axis=0),
    ])
    part = jnp.concatenate([rows, jnp.zeros((3, _H), jnp.float32)], axis=0)

    @pl.when(i == 0)
    def _():
        out_ref[...] = jnp.zeros_like(out_ref)

    out_ref[...] += part


def _main_body(g_ref, p_ref, sums_ref, w2_ref, gb1_ref, mx_ref, mn_ref, s2_ref):
    i = pl.program_id(0)
    inv_cnt = 1.0 / float(_N * _K)
    s = sums_ref[...]
    mean1 = (s[0] + _K * s[3]) * inv_cnt
    ex2 = (s[1] + 2.0 * s[2] + _K * s[4]) * inv_cnt
    var1 = ex2 - mean1 * mean1
    inv1 = lax.rsqrt(var1 + _EPS)
    sc1 = gb1_ref[0] * inv1
    t1 = gb1_ref[1] - mean1 * sc1

    g = g_ref[...]
    h1 = g + p_ref[...][:, None, :]
    z = h1 * sc1 + t1
    u = jnp.where(z >= 0.0, z, 0.2 * z)
    u2 = u.reshape(_TILE * _K, _H)
    h2 = jnp.dot(u2, w2_ref[...], preferred_element_type=jnp.float32)
    s2 = jnp.sum(h2, axis=0)
    s2q = jnp.sum(h2 * h2, axis=0)
    h23 = h2.reshape(_TILE, _K, _O)
    mx_ref[...] = jnp.max(h23, axis=1)
    mn_ref[...] = jnp.min(h23, axis=1)
    part = jnp.concatenate(
        [s2[None], s2q[None], jnp.zeros((6, _O), jnp.float32)], axis=0)

    @pl.when(i == 0)
    def _():
        s2_ref[...] = jnp.zeros_like(s2_ref)

    s2_ref[...] += part


def _final_body(mx_ref, mn_ref, s2_ref, gb2_ref, out_ref):
    inv_cnt = 1.0 / float(_N * _K)
    s = s2_ref[...]
    mean2 = s[0] * inv_cnt
    var2 = s[1] * inv_cnt - mean2 * mean2
    inv2 = lax.rsqrt(var2 + _EPS)
    sc2 = gb2_ref[0] * inv2
    t2 = gb2_ref[1] - mean2 * sc2
    pick = jnp.where(sc2 >= 0.0, mx_ref[...], mn_ref[...])
    z = pick * sc2 + t2
    out_ref[...] = jnp.where(z >= 0.0, z, 0.2 * z)


def kernel(x, W1, gamma1, beta1, W2, gamma2, beta2, idx):
    x0 = x.reshape(_N, _C)
    idxg = idx.reshape(_N * _K).astype(jnp.int32)
    gb1 = jnp.stack([gamma1, beta1])
    gb2 = jnp.stack([gamma2, beta2])

    A, P = pl.pallas_call(
        _prep_body,
        grid=(_NT,),
        in_specs=[
            pl.BlockSpec((_TILE, _C), lambda i: (i, 0)),
            pl.BlockSpec((2 * _C, _H), lambda i: (0, 0)),
        ],
        out_specs=[
            pl.BlockSpec((_TILE, _H), lambda i: (i, 0)),
            pl.BlockSpec((_TILE, _H), lambda i: (i, 0)),
        ],
        out_shape=[
            jax.ShapeDtypeStruct((_N, _H), jnp.float32),
            jax.ShapeDtypeStruct((_N, _H), jnp.float32),
        ],
    )(x0, W1)

    gathered = _sc_gather_fn()(A, idxg)
    g3 = gathered.reshape(_N, _K, _H)

    sums1 = pl.pallas_call(
        _stats_body,
        grid=(_NT,),
        in_specs=[
            pl.BlockSpec((_TILE, _K, _H), lambda i: (i, 0, 0)),
            pl.BlockSpec((_TILE, _H), lambda i: (i, 0)),
        ],
        out_specs=pl.BlockSpec((8, _H), lambda i: (0, 0)),
        out_shape=jax.ShapeDtypeStruct((8, _H), jnp.float32),
    )(g3, P)

    mx, mn, sums2 = pl.pallas_call(
        _main_body,
        grid=(_NT,),
        in_specs=[
            pl.BlockSpec((_TILE, _K, _H), lambda i: (i, 0, 0)),
            pl.BlockSpec((_TILE, _H), lambda i: (i, 0)),
            pl.BlockSpec((8, _H), lambda i: (0, 0)),
            pl.BlockSpec((_H, _O), lambda i: (0, 0)),
            pl.BlockSpec((2, _H), lambda i: (0, 0)),
        ],
        out_specs=[
            pl.BlockSpec((_TILE, _O), lambda i: (i, 0)),
            pl.BlockSpec((_TILE, _O), lambda i: (i, 0)),
            pl.BlockSpec((8, _O), lambda i: (0, 0)),
        ],
        out_shape=[
            jax.ShapeDtypeStruct((_N, _O), jnp.float32),
            jax.ShapeDtypeStruct((_N, _O), jnp.float32),
            jax.ShapeDtypeStruct((8, _O), jnp.float32),
        ],
    )(g3, P, sums1, W2, gb1)

    out = pl.pallas_call(
        _final_body,
        grid=(_NT,),
        in_specs=[
            pl.BlockSpec((_TILE, _O), lambda i: (i, 0)),
            pl.BlockSpec((_TILE, _O), lambda i: (i, 0)),
            pl.BlockSpec((8, _O), lambda i: (0, 0)),
            pl.BlockSpec((2, _O), lambda i: (0, 0)),
        ],
        out_specs=pl.BlockSpec((_TILE, _O), lambda i: (i, 0)),
        out_shape=jax.ShapeDtypeStruct((_N, _O), jnp.float32),
    )(mx, mn, sums2, gb2)

    return (out.reshape(1, _N, _O), idx)


# speedup vs baseline: 7.1712x; 1.5507x over previous
import functools

import jax
import jax.numpy as jnp
from jax import lax
from jax.experimental import pallas as pl
from jax.experimental.pallas import tpu as pltpu
from jax.experimental.pallas import tpu_sc as plsc

_N = 10000
_K = 16
_C = 128
_H = 128
_O = 128
_EPS = 1e-5

_TP = 2000
_NTP = _N // _TP
_TILE = 400
_NT = _N // _TILE

_NW = 32
_PER_W = _N * _K // _NW
_CH = 200
_NCH = _PER_W // _CH


def _prep_body(x_ref, w1_ref, a_ref, p_ref):
    xb = x_ref[...]
    wa = w1_ref[:_C, :]
    wb = w1_ref[_C:, :]
    a_ref[...] = jnp.dot(xb, wa, preferred_element_type=jnp.float32)
    p_ref[...] = jnp.dot(xb, wb - wa, preferred_element_type=jnp.float32)


@functools.cache
def _sc_gather_fn():
    mesh = plsc.VectorSubcoreMesh(core_axis_name="c", subcore_axis_name="s")

    @functools.partial(
        pl.kernel,
        mesh=mesh,
        out_type=jax.ShapeDtypeStruct((_N * _K, _H), jnp.float32),
        scratch_types=[
            pltpu.VMEM((_PER_W,), jnp.int32),
            pltpu.VMEM((_CH, _H), jnp.float32),
            pltpu.VMEM((_CH, _H), jnp.float32),
            pltpu.SemaphoreType.DMA,
            pltpu.SemaphoreType.DMA,
            pltpu.SemaphoreType.DMA,
            pltpu.SemaphoreType.DMA,
        ],
    )
    def _sc_gather(table_hbm, idx_hbm, out_hbm, idx_v, buf0, buf1,
                   gsem0, gsem1, ssem0, ssem1):
        wid = lax.axis_index("s") * 2 + lax.axis_index("c")
        base = wid * _PER_W
        pltpu.sync_copy(idx_hbm.at[pl.ds(base, _PER_W)], idx_v)
        bufs = (buf0, buf1)
        gsems = (gsem0, gsem1)
        ssems = (ssem0, ssem1)
        gathers = [None] * _NCH
        scatters = [None] * _NCH

        def start_gather(i):
            return pltpu.async_copy(
                table_hbm.at[idx_v.at[pl.ds(i * _CH, _CH)]],
                bufs[i % 2], gsems[i % 2])

        gathers[0] = start_gather(0)
        for i in range(_NCH):
            if i + 1 < _NCH:
                if i >= 1:
                    scatters[i - 1].wait()
                gathers[i + 1] = start_gather(i + 1)
            gathers[i].wait()
            scatters[i] = pltpu.async_copy(
                bufs[i % 2], out_hbm.at[pl.ds(base + i * _CH, _CH)],
                ssems[i % 2])
        scatters[_NCH - 2].wait()
        scatters[_NCH - 1].wait()

    return _sc_gather


def _stats_body(g_ref, p_ref, out_ref):
    i = pl.program_id(0)
    g = g_ref[...]
    p = p_ref[...]
    t = jnp.sum(g, axis=1)
    rows = jnp.stack([
        jnp.sum(t, axis=0),
        jnp.sum(g * g, axis=(0, 1)),
        jnp.sum(p * t, axis=0),
        jnp.sum(p, axis=0),
        jnp.sum(p * p, axis=0),
    ])
    part = jnp.concatenate([rows, jnp.zeros((3, _H), jnp.float32)], axis=0)

    @pl.when(i == 0)
    def _():
        out_ref[...] = jnp.zeros_like(out_ref)

    out_ref[...] += part


def _main_body(g_ref, p_ref, sums_ref, w2_ref, gb1_ref, mx_ref, s2_ref):
    i = pl.program_id(0)
    inv_cnt = 1.0 / float(_N * _K)
    s = sums_ref[...]
    mean1 = (s[0] + _K * s[3]) * inv_cnt
    ex2 = (s[1] + 2.0 * s[2] + _K * s[4]) * inv_cnt
    var1 = ex2 - mean1 * mean1
    inv1 = lax.rsqrt(var1 + _EPS)
    sc1 = gb1_ref[0] * inv1
    t1 = gb1_ref[1] - mean1 * sc1

    g = g_ref[...]
    q = p_ref[...] * sc1 + t1
    z = g * sc1 + q[:, None, :]
    u = jnp.where(z >= 0.0, z, 0.2 * z)
    u2 = u.reshape(_TILE * _K, _H)
    h2 = jnp.dot(u2, w2_ref[...], preferred_element_type=jnp.float32)
    s2 = jnp.sum(h2, axis=0)
    s2q = jnp.sum(h2 * h2, axis=0)
    mx_ref[...] = jnp.max(h2.reshape(_TILE, _K, _O), axis=1)
    part = jnp.concatenate(
        [s2[None], s2q[None], jnp.zeros((6, _O), jnp.float32)], axis=0)

    @pl.when(i == 0)
    def _():
        s2_ref[...] = jnp.zeros_like(s2_ref)

    s2_ref[...] += part


def _final_body(mx_ref, s2_ref, gb2_ref, out_ref):
    inv_cnt = 1.0 / float(_N * _K)
    s = s2_ref[...]
    mean2 = s[0] * inv_cnt
    var2 = s[1] * inv_cnt - mean2 * mean2
    inv2 = lax.rsqrt(var2 + _EPS)
    sc2 = gb2_ref[0] * inv2
    t2 = gb2_ref[1] - mean2 * sc2
    z = mx_ref[...] * sc2 + t2
    out_ref[...] = jnp.where(z >= 0.0, z, 0.2 * z)


def kernel(x, W1, gamma1, beta1, W2, gamma2, beta2, idx):
    x0 = x.reshape(_N, _C)
    idxg = idx.reshape(_N * _K).astype(jnp.int32)
    gb1 = jnp.stack([gamma1, beta1])
    gb2 = jnp.stack([gamma2, beta2])

    A, P = pl.pallas_call(
        _prep_body,
        grid=(_NTP,),
        in_specs=[
            pl.BlockSpec((_TP, _C), lambda i: (i, 0)),
            pl.BlockSpec((2 * _C, _H), lambda i: (0, 0)),
        ],
        out_specs=[
            pl.BlockSpec((_TP, _H), lambda i: (i, 0)),
            pl.BlockSpec((_TP, _H), lambda i: (i, 0)),
        ],
        out_shape=[
            jax.ShapeDtypeStruct((_N, _H), jnp.float32),
            jax.ShapeDtypeStruct((_N, _H), jnp.float32),
        ],
    )(x0, W1)

    gathered = _sc_gather_fn()(A, idxg)
    g3 = gathered.reshape(_N, _K, _H)

    sums1 = pl.pallas_call(
        _stats_body,
        grid=(_NT,),
        in_specs=[
            pl.BlockSpec((_TILE, _K, _H), lambda i: (i, 0, 0)),
            pl.BlockSpec((_TILE, _H), lambda i: (i, 0)),
        ],
        out_specs=pl.BlockSpec((8, _H), lambda i: (0, 0)),
        out_shape=jax.ShapeDtypeStruct((8, _H), jnp.float32),
    )(g3, P)

    mx, sums2 = pl.pallas_call(
        _main_body,
        grid=(_NT,),
        in_specs=[
            pl.BlockSpec((_TILE, _K, _H), lambda i: (i, 0, 0)),
            pl.BlockSpec((_TILE, _H), lambda i: (i, 0)),
            pl.BlockSpec((8, _H), lambda i: (0, 0)),
            pl.BlockSpec((_H, _O), lambda i: (0, 0)),
            pl.BlockSpec((2, _H), lambda i: (0, 0)),
        ],
        out_specs=[
            pl.BlockSpec((_TILE, _O), lambda i: (i, 0)),
            pl.BlockSpec((8, _O), lambda i: (0, 0)),
        ],
        out_shape=[
            jax.ShapeDtypeStruct((_N, _O), jnp.float32),
            jax.ShapeDtypeStruct((8, _O), jnp.float32),
        ],
    )(g3, P, sums1, W2, gb1)

    out = pl.pallas_call(
        _final_body,
        grid=(_NTP,),
        in_specs=[
            pl.BlockSpec((_TP, _O), lambda i: (i, 0)),
            pl.BlockSpec((8, _O), lambda i: (0, 0)),
            pl.BlockSpec((2, _O), lambda i: (0, 0)),
        ],
        out_specs=pl.BlockSpec((_TP, _O), lambda i: (i, 0)),
        out_shape=jax.ShapeDtypeStruct((_N, _O), jnp.float32),
    )(mx, sums2, gb2)

    return (out.reshape(1, _N, _O), idx)
